# baseline (speedup 1.0000x reference)
import functools

import jax
import jax.numpy as jnp
from jax import lax
from jax.experimental import pallas as pl
from jax.experimental.pallas import tpu as pltpu
from jax.experimental.pallas import tpu_sc as plsc

_NC = 2
_NS = 16
_NW = _NC * _NS
_CH = 200


def _sc_copy_pair(t, w):
    m, n = t.shape
    n_chunks = m // _CH
    full_iters = n_chunks // _NW
    extra = n_chunks - full_iters * _NW
    mesh = plsc.VectorSubcoreMesh(core_axis_name="c", subcore_axis_name="s")

    @functools.partial(
        pl.kernel,
        out_type=(
            jax.ShapeDtypeStruct((m, n), t.dtype),
            jax.ShapeDtypeStruct((m, n), w.dtype),
        ),
        mesh=mesh,
        scratch_types=[
            pltpu.VMEM((_CH, n), t.dtype),
            pltpu.VMEM((_CH, n), t.dtype),
            pltpu.VMEM((_CH, n), w.dtype),
            pltpu.VMEM((_CH, n), w.dtype),
            pltpu.SemaphoreType.DMA((2,)),
            pltpu.SemaphoreType.DMA((2,)),
            pltpu.SemaphoreType.DMA((2,)),
            pltpu.SemaphoreType.DMA((2,)),
        ],
    )
    def body(t_in, w_in, t_out, w_out, bt0, bt1, bw0, bw1,
             s_in_t, s_in_w, s_out_t, s_out_w):
        wid = lax.axis_index("s") * _NC + lax.axis_index("c")
        bufs_t = (bt0, bt1)
        bufs_w = (bw0, bw1)
        outs = {}

        def start_iter(k):
            b = k % 2
            sl = pl.ds((wid + _NW * k) * _CH, _CH)
            hit = pltpu.async_copy(t_in.at[sl], bufs_t[b], s_in_t.at[b])
            hiw = pltpu.async_copy(w_in.at[sl], bufs_w[b], s_in_w.at[b])
            hit.wait()
            hiw.wait()
            outs[k] = (
                pltpu.async_copy(bufs_t[b], t_out.at[sl], s_out_t.at[b]),
                pltpu.async_copy(bufs_w[b], w_out.at[sl], s_out_w.at[b]),
            )

        for k in range(full_iters):
            if k >= 2:
                outs[k - 2][0].wait()
                outs[k - 2][1].wait()
            start_iter(k)

        if extra:
            outs[full_iters - 2][0].wait()
            outs[full_iters - 2][1].wait()

            @pl.when(wid < extra)
            def _():
                b = full_iters % 2
                sl = pl.ds((wid + _NW * full_iters) * _CH, _CH)
                hit = pltpu.async_copy(t_in.at[sl], bufs_t[b], s_in_t.at[b])
                hiw = pltpu.async_copy(w_in.at[sl], bufs_w[b], s_in_w.at[b])
                hit.wait()
                hiw.wait()
                hot = pltpu.async_copy(bufs_t[b], t_out.at[sl], s_out_t.at[b])
                how = pltpu.async_copy(bufs_w[b], w_out.at[sl], s_out_w.at[b])
                hot.wait()
                how.wait()
        else:
            outs[full_iters - 2][0].wait()
            outs[full_iters - 2][1].wait()

        outs[full_iters - 1][0].wait()
        outs[full_iters - 1][1].wait()

    return body(t, w)


def kernel(bbox_targets, bbox_weights, labels):
    del labels
    return _sc_copy_pair(bbox_targets, bbox_weights)


# speedup vs baseline: 1.0516x; 1.0516x over previous
import functools

import jax
import jax.numpy as jnp
from jax import lax
from jax.experimental import pallas as pl
from jax.experimental.pallas import tpu as pltpu
from jax.experimental.pallas import tpu_sc as plsc

_BR = 8000

_NC = 2
_NS = 16
_NW = _NC * _NS
_SUB = 1000


def _tc_copy_kernel(t_in, t_out):
    t_out[...] = t_in[...]


def _tc_copy(x):
    m, n = x.shape
    spec = pl.BlockSpec((_BR, n), lambda i: (i, 0))
    return pl.pallas_call(
        _tc_copy_kernel,
        grid=(m // _BR,),
        in_specs=[spec],
        out_specs=spec,
        out_shape=jax.ShapeDtypeStruct((m, n), x.dtype),
    )(x)


def _sc_copy(x):
    m, n = x.shape
    chunk = ((m + _NW - 1) // _NW + 7) // 8 * 8
    last = m - (_NW - 1) * chunk
    n_full, tail = divmod(chunk, _SUB)
    n_full_last, tail_last = divmod(last, _SUB)
    mesh = plsc.VectorSubcoreMesh(core_axis_name="c", subcore_axis_name="s")

    @functools.partial(
        pl.kernel,
        out_type=jax.ShapeDtypeStruct((m, n), x.dtype),
        mesh=mesh,
        scratch_types=[pltpu.VMEM((_SUB, n), x.dtype)],
    )
    def body(in_hbm, out_hbm, buf):
        wid = lax.axis_index("s") * _NC + lax.axis_index("c")
        base = wid * chunk

        def move(start, size):
            sl = pl.ds(start, size)
            pltpu.sync_copy(in_hbm.at[sl], buf.at[pl.ds(0, size)])
            pltpu.sync_copy(buf.at[pl.ds(0, size)], out_hbm.at[sl])

        @pl.when(wid < _NW - 1)
        def _():
            def step(j, _):
                move(base + j * _SUB, _SUB)
                return ()
            lax.fori_loop(0, n_full, step, ())
            if tail:
                move(base + n_full * _SUB, tail)

        @pl.when(wid == _NW - 1)
        def _():
            def step(j, _):
                move(base + j * _SUB, _SUB)
                return ()
            lax.fori_loop(0, n_full_last, step, ())
            if tail_last:
                move(base + n_full_last * _SUB, tail_last)

    return body(x)


def kernel(bbox_targets, bbox_weights, labels):
    del labels
    w = _sc_copy(bbox_weights)
    t = _tc_copy(bbox_targets)
    return (t, w)
